# baseline (speedup 1.0000x reference)
import jax
import jax.numpy as jnp
from jax.experimental import pallas as pl
from jax.experimental.pallas import tpu as pltpu

N = 8192
E = 8192
BM = 256
BM3 = 512
_EPS = 1e-5

_C1 = 0
_YW = 48
_ZC = 80
_Z1 = 96


def _dot(a, b):
    return jax.lax.dot_general(
        a, b, (((1,), (0,)), ((), ())),
        precision=jax.lax.Precision.DEFAULT,
        preferred_element_type=jnp.float32)


def _bn_relu_max(zc, g, be):
    m = jnp.mean(zc, axis=0, keepdims=True)
    v = jnp.mean(jnp.square(zc), axis=0, keepdims=True) - jnp.square(m)
    zp = jax.nn.relu((zc - m) * jax.lax.rsqrt(v + _EPS) * g + be)
    return jnp.max(zp, axis=1, keepdims=True)


def _ka_body(xn_ref, xe_ref, w1t_ref, b1_ref, w2t_ref, b2_ref,
             hw1t_ref, hb1_ref, g1_ref, be1_ref,
             a_ref, l1_ref,
             h_ref, urz_ref, scr):
    p = pl.program_id(0)
    i = pl.program_id(1)
    rows = pl.ds(i * BM, BM)

    @pl.when((p == 0) & (i == 0))
    def _():
        scr[:, _C1:_C1 + 32] = _dot(xn_ref[:], w1t_ref[:])
        scr[:, _C1 + 32:_C1 + 40] = _dot(xe_ref[:], hw1t_ref[:]) + hb1_ref[:]
        scr[:, _C1 + 40:_C1 + 41] = jnp.ones((E, 1), jnp.float32)

    @pl.when(p == 0)
    def _():
        y1 = jax.nn.relu(_dot(a_ref[:], scr[:, _C1:_C1 + 32]) + b1_ref[:])
        scr[rows, _YW:_YW + 32] = _dot(y1, w2t_ref[:])
        scr[rows, _ZC:_ZC + 9] = _dot(l1_ref[:], scr[:, _C1 + 32:_C1 + 41])

    @pl.when((p == 1) & (i == 0))
    def _():
        scr[:, _Z1:_Z1 + 1] = _bn_relu_max(
            scr[:, _ZC:_ZC + 8], g1_ref[:], be1_ref[:])

    @pl.when(p == 1)
    def _():
        h_ref[:] = jax.nn.relu(_dot(a_ref[:], scr[:, _YW:_YW + 32])
                               + b2_ref[:])
        urz_ref[:, 0:1] = _dot(l1_ref[:], scr[:, _Z1:_Z1 + 1])
        urz_ref[:, 1:2] = scr[rows, _ZC + 8:_ZC + 9]
        urz_ref[:, 2:3] = scr[rows, _Z1:_Z1 + 1]


def _kb_body(urz_ref, hw2t_ref, hb2_ref, g2_ref, be2_ref,
             ehwt_ref, ehb_ref, nhwt_ref, nhb_ref,
             b1m_ref, h_ref,
             hcat_ref, np_ref, ep_ref, zh_scr):
    i = pl.program_id(0)

    @pl.when(i == 0)
    def _():
        zc2 = (urz_ref[:, 0:1] * hw2t_ref[:]
               + urz_ref[:, 1:2] * hb2_ref[:])
        z2 = _bn_relu_max(zc2, g2_ref[:], be2_ref[:])
        zh_scr[:, 0:1] = urz_ref[:, 2:3]
        zh_scr[:, 1:2] = z2
        ep_ref[:] = jax.nn.sigmoid(_dot(zh_scr[:], ehwt_ref[:])
                                   + ehb_ref[:])

    hcat_ref[:, :32] = h_ref[:]
    hcat_ref[:, 32:34] = _dot(b1m_ref[:], zh_scr[:])
    np_ref[:] = jax.nn.sigmoid(_dot(hcat_ref[:], nhwt_ref[:]) + nhb_ref[:])


def _full(shape):
    return pl.BlockSpec(shape, lambda *_: (0,) * len(shape))


def kernel(X_n, X_e, A_tilde, L1_tilde, B1, gnn_w1, gnn_b1, gnn_w2, gnn_b2,
           hosc1_w, hosc1_b, hosc1_g, hosc1_be, hosc2_w, hosc2_b, hosc2_g,
           hosc2_be, nh_w, nh_b, eh_w, eh_b):
    f32 = jnp.float32

    h, urz = pl.pallas_call(
        _ka_body,
        grid=(2, N // BM),
        in_specs=[_full((N, 128)), _full((E, 16)), _full((128, 32)),
                  _full((1, 32)), _full((32, 32)), _full((1, 32)),
                  _full((16, 8)), _full((1, 8)), _full((1, 8)),
                  _full((1, 8)),
                  pl.BlockSpec((BM, N), lambda p, i: (i, 0)),
                  pl.BlockSpec((BM, E), lambda p, i: (i, 0))],
        out_specs=[pl.BlockSpec((BM, 32), lambda p, i: (i, 0)),
                   pl.BlockSpec((BM, 3), lambda p, i: (i, 0))],
        out_shape=[jax.ShapeDtypeStruct((N, 32), f32),
                   jax.ShapeDtypeStruct((E, 3), f32)],
        scratch_shapes=[pltpu.VMEM((N, 128), f32)],
    )(X_n, X_e, gnn_w1.T, gnn_b1.reshape(1, -1), gnn_w2.T,
      gnn_b2.reshape(1, -1), hosc1_w.T, hosc1_b.reshape(1, -1),
      hosc1_g.reshape(1, -1), hosc1_be.reshape(1, -1), A_tilde, L1_tilde)

    hcat, np_, ep = pl.pallas_call(
        _kb_body,
        grid=(N // BM3,),
        in_specs=[_full((E, 3)), _full((1, 8)), _full((1, 8)),
                  _full((1, 8)), _full((1, 8)), _full((2, 1)),
                  _full((1, 1)), _full((34, 1)), _full((1, 1)),
                  pl.BlockSpec((BM3, E), lambda i: (i, 0)),
                  pl.BlockSpec((BM3, 32), lambda i: (i, 0))],
        out_specs=[pl.BlockSpec((BM3, 34), lambda i: (i, 0)),
                   pl.BlockSpec((BM3, 1), lambda i: (i, 0)),
                   _full((E, 1))],
        out_shape=[jax.ShapeDtypeStruct((N, 34), f32),
                   jax.ShapeDtypeStruct((N, 1), f32),
                   jax.ShapeDtypeStruct((E, 1), f32)],
        scratch_shapes=[pltpu.VMEM((E, 2), f32)],
    )(urz, hosc2_w.T, hosc2_b.reshape(1, -1), hosc2_g.reshape(1, -1),
      hosc2_be.reshape(1, -1), eh_w.T, eh_b.reshape(1, -1), nh_w.T,
      nh_b.reshape(1, -1), B1, h)

    return np_[:, 0], ep[:, 0], hcat


# speedup vs baseline: 1.0162x; 1.0162x over previous
import jax
import jax.numpy as jnp
from jax.experimental import pallas as pl
from jax.experimental.pallas import tpu as pltpu

N = 8192
E = 8192
BM = 512
_EPS = 1e-5


def _dot(a, b):
    return jax.lax.dot_general(
        a, b, (((1,), (0,)), ((), ())),
        precision=jax.lax.Precision.DEFAULT,
        preferred_element_type=jnp.float32)


def _bn_relu_max(zc, g, be):
    m = jnp.mean(zc, axis=0, keepdims=True)
    v = jnp.mean(jnp.square(zc), axis=0, keepdims=True) - jnp.square(m)
    zp = jax.nn.relu((zc - m) * jax.lax.rsqrt(v + _EPS) * g + be)
    return jnp.max(zp, axis=1, keepdims=True)


def _k1_body(xn_ref, w1t_ref, b1_ref, w2t_ref, a_ref, yw_ref, xw_scr):
    @pl.when(pl.program_id(0) == 0)
    def _():
        xw_scr[:] = _dot(xn_ref[:], w1t_ref[:])

    y1 = jax.nn.relu(_dot(a_ref[:], xw_scr[:]) + b1_ref[:])
    yw_ref[:] = _dot(y1, w2t_ref[:])


def _k2_body(xe_ref, hw1t_ref, hb1_ref, l1_ref, zca_ref, zt_scr):
    @pl.when(pl.program_id(0) == 0)
    def _():
        zt_scr[:, :8] = _dot(xe_ref[:], hw1t_ref[:]) + hb1_ref[:]
        zt_scr[:, 8:9] = jnp.ones((E, 1), jnp.float32)

    zca_ref[:] = _dot(l1_ref[:], zt_scr[:])


def _k3_body(yw_ref, b2_ref, a_ref, h_ref):
    h_ref[:] = jax.nn.relu(_dot(a_ref[:], yw_ref[:]) + b2_ref[:])


def _k4_body(zca_ref, g1_ref, be1_ref, l1_ref, urz_ref, z1_scr):
    i = pl.program_id(0)

    @pl.when(i == 0)
    def _():
        z1_scr[:] = _bn_relu_max(zca_ref[:, :8], g1_ref[:], be1_ref[:])

    rows = pl.ds(i * BM, BM)
    urz_ref[:, 0:1] = _dot(l1_ref[:], z1_scr[:])
    urz_ref[:, 1:2] = zca_ref[rows, 8:9]
    urz_ref[:, 2:3] = z1_scr[rows, :]


def _k5_body(urz_ref, hw2t_ref, hb2_ref, g2_ref, be2_ref,
             ehwt_ref, ehb_ref, nhwt_ref, nhb_ref,
             b1m_ref, h_ref,
             hcat_ref, np_ref, ep_ref, zh_scr):
    @pl.when(pl.program_id(0) == 0)
    def _():
        zc2 = (urz_ref[:, 0:1] * hw2t_ref[:]
               + urz_ref[:, 1:2] * hb2_ref[:])
        z2 = _bn_relu_max(zc2, g2_ref[:], be2_ref[:])
        zh_scr[:, 0:1] = urz_ref[:, 2:3]
        zh_scr[:, 1:2] = z2
        ep_ref[:] = jax.nn.sigmoid(_dot(zh_scr[:], ehwt_ref[:])
                                   + ehb_ref[:])

    hcat_ref[:, :32] = h_ref[:]
    hcat_ref[:, 32:34] = _dot(b1m_ref[:], zh_scr[:])
    np_ref[:] = jax.nn.sigmoid(_dot(hcat_ref[:], nhwt_ref[:]) + nhb_ref[:])


def _full(shape):
    return pl.BlockSpec(shape, lambda *_: (0,) * len(shape))


def _rows(width):
    return pl.BlockSpec((BM, width), lambda i: (i, 0))


def kernel(X_n, X_e, A_tilde, L1_tilde, B1, gnn_w1, gnn_b1, gnn_w2, gnn_b2,
           hosc1_w, hosc1_b, hosc1_g, hosc1_be, hosc2_w, hosc2_b, hosc2_g,
           hosc2_be, nh_w, nh_b, eh_w, eh_b):
    f32 = jnp.float32
    grid = (N // BM,)

    yw = pl.pallas_call(
        _k1_body,
        grid=grid,
        in_specs=[_full((N, 128)), _full((128, 32)), _full((1, 32)),
                  _full((32, 32)), _rows(N)],
        out_specs=_rows(32),
        out_shape=jax.ShapeDtypeStruct((N, 32), f32),
        scratch_shapes=[pltpu.VMEM((N, 32), f32)],
    )(X_n, gnn_w1.T, gnn_b1.reshape(1, -1), gnn_w2.T, A_tilde)

    zca = pl.pallas_call(
        _k2_body,
        grid=grid,
        in_specs=[_full((E, 16)), _full((16, 8)), _full((1, 8)), _rows(E)],
        out_specs=_rows(9),
        out_shape=jax.ShapeDtypeStruct((E, 9), f32),
        scratch_shapes=[pltpu.VMEM((E, 9), f32)],
    )(X_e, hosc1_w.T, hosc1_b.reshape(1, -1), L1_tilde)

    h = pl.pallas_call(
        _k3_body,
        grid=grid,
        in_specs=[_full((N, 32)), _full((1, 32)), _rows(N)],
        out_specs=_rows(32),
        out_shape=jax.ShapeDtypeStruct((N, 32), f32),
    )(yw, gnn_b2.reshape(1, -1), A_tilde)

    urz = pl.pallas_call(
        _k4_body,
        grid=grid,
        in_specs=[_full((E, 9)), _full((1, 8)), _full((1, 8)), _rows(E)],
        out_specs=_rows(3),
        out_shape=jax.ShapeDtypeStruct((E, 3), f32),
        scratch_shapes=[pltpu.VMEM((E, 1), f32)],
    )(zca, hosc1_g.reshape(1, -1), hosc1_be.reshape(1, -1), L1_tilde)

    hcat, np_, ep = pl.pallas_call(
        _k5_body,
        grid=grid,
        in_specs=[_full((E, 3)), _full((1, 8)), _full((1, 8)),
                  _full((1, 8)), _full((1, 8)), _full((2, 1)),
                  _full((1, 1)), _full((34, 1)), _full((1, 1)),
                  _rows(E), _rows(32)],
        out_specs=[_rows(34), _rows(1), _full((E, 1))],
        out_shape=[jax.ShapeDtypeStruct((N, 34), f32),
                   jax.ShapeDtypeStruct((N, 1), f32),
                   jax.ShapeDtypeStruct((E, 1), f32)],
        scratch_shapes=[pltpu.VMEM((E, 2), f32)],
    )(urz, hosc2_w.T, hosc2_b.reshape(1, -1), hosc2_g.reshape(1, -1),
      hosc2_be.reshape(1, -1), eh_w.T, eh_b.reshape(1, -1), nh_w.T,
      nh_b.reshape(1, -1), B1, h)

    return np_[:, 0], ep[:, 0], hcat
